# baseline (speedup 1.0000x reference)
import functools

import jax
import jax.numpy as jnp
from jax import lax
from jax.experimental import pallas as pl
from jax.experimental.pallas import tpu as pltpu
from jax.experimental.pallas import tpu_sc as plsc

B, C, H, W = 4, 96, 384, 384
CP = 128
N, S = 4096, 16
QB = N * S
NC, NS, L = 2, 16, 16
NW = NC * NS
QPW = QB // NW
P = 128
NCHUNK = QPW // P
CL = C // L

HG = 8
NH = H // HG


def _tr_body(x_ref, o_ref):
    row = lax.broadcasted_iota(jnp.int32, (C, CP), 0)
    col = lax.broadcasted_iota(jnp.int32, (C, CP), 1)
    eye = (row == col).astype(jnp.float32)
    for j in range(HG):
        a = x_ref[:, j, :]
        t = lax.dot_general(a, eye, (((0,), (0,)), ((), ())),
                            preferred_element_type=jnp.float32)
        o_ref[pl.ds(j * W, W), :] = t


@jax.jit
def _transpose_batch(xb):
    return pl.pallas_call(
        _tr_body,
        out_shape=jax.ShapeDtypeStruct((H * W, CP), jnp.float32),
        grid=(NH,),
        in_specs=[pl.BlockSpec((C, HG, W), lambda i: (0, i, 0))],
        out_specs=pl.BlockSpec((HG * W, CP), lambda i: (i, 0)),
    )(xb)


def _sc_body(xt, gx, gy, out,
             gx_v, gy_v,
             w00_v, w01_v, w10_v, w11_v,
             i00_v, i01_v, i10_v, i11_v,
             r00, r01, r10, r11, out_v, sem):
    cid = lax.axis_index("c")
    sid = lax.axis_index("s")
    wid = sid * NC + cid
    qbase = wid * QPW

    pltpu.sync_copy(gx.at[pl.ds(qbase, QPW)], gx_v)
    pltpu.sync_copy(gy.at[pl.ds(qbase, QPW)], gy_v)

    def chunk(k, _):
        off = k * P

        def grp(j, _):
            sl = pl.ds(off + j * L, L)
            dst = pl.ds(j * L, L)
            px = (gx_v[sl] + 1.0) * (0.5 * (W - 1))
            py = (gy_v[sl] + 1.0) * (0.5 * (H - 1))
            px = jnp.minimum(jnp.maximum(px, 0.0), float(W - 1))
            py = jnp.minimum(jnp.maximum(py, 0.0), float(H - 1))
            x0 = px.astype(jnp.int32)
            y0 = py.astype(jnp.int32)
            fx = px - x0.astype(jnp.float32)
            fy = py - y0.astype(jnp.float32)
            dx = jnp.minimum(x0 + 1, W - 1) - x0
            dy = jnp.minimum(y0 + 1, H - 1) - y0
            i00 = y0 * W + x0
            i10 = i00 + dy * W
            gx1 = 1.0 - fx
            gy1 = 1.0 - fy
            w00_v[dst] = gx1 * gy1
            w01_v[dst] = fx * gy1
            w10_v[dst] = gx1 * fy
            w11_v[dst] = fx * fy
            i00_v[dst] = i00
            i01_v[dst] = i00 + dx
            i10_v[dst] = i10
            i11_v[dst] = i10 + dx
            return 0

        lax.fori_loop(0, P // L, grp, 0)

        cp0 = pltpu.async_copy(xt.at[i00_v], r00, sem)
        cp1 = pltpu.async_copy(xt.at[i01_v], r01, sem)
        cp2 = pltpu.async_copy(xt.at[i10_v], r10, sem)
        cp3 = pltpu.async_copy(xt.at[i11_v], r11, sem)
        cp0.wait()
        cp1.wait()
        cp2.wait()
        cp3.wait()

        def blend(g, _):
            gsl = pl.ds(g * L, L)
            w00 = w00_v[gsl]
            w01 = w01_v[gsl]
            w10 = w10_v[gsl]
            w11 = w11_v[gsl]
            for ii in range(L):
                i = g * L + ii
                a = w00[ii]
                b = w01[ii]
                c = w10[ii]
                d = w11[ii]
                for cc in range(CL):
                    sl = pl.ds(cc * L, L)
                    out_v[i, sl] = (a * r00[i, sl] + b * r01[i, sl]
                                    + c * r10[i, sl] + d * r11[i, sl])
            return 0

        lax.fori_loop(0, P // L, blend, 0)

        pltpu.sync_copy(out_v, out.at[pl.ds(qbase + off, P)])
        return 0

    lax.fori_loop(0, NCHUNK, chunk, 0)


def _sc_gather(xt, gx, gy):
    mesh = plsc.VectorSubcoreMesh(core_axis_name="c", subcore_axis_name="s")
    return pl.kernel(
        _sc_body,
        out_type=jax.ShapeDtypeStruct((QB, CP), jnp.float32),
        mesh=mesh,
        compiler_params=pltpu.CompilerParams(use_tc_tiling_on_sc=True),
        scratch_types=[
            pltpu.VMEM((QPW,), jnp.float32),
            pltpu.VMEM((QPW,), jnp.float32),
            pltpu.VMEM((P,), jnp.float32),
            pltpu.VMEM((P,), jnp.float32),
            pltpu.VMEM((P,), jnp.float32),
            pltpu.VMEM((P,), jnp.float32),
            pltpu.VMEM((P,), jnp.int32),
            pltpu.VMEM((P,), jnp.int32),
            pltpu.VMEM((P,), jnp.int32),
            pltpu.VMEM((P,), jnp.int32),
            pltpu.VMEM((P, CP), jnp.float32),
            pltpu.VMEM((P, CP), jnp.float32),
            pltpu.VMEM((P, CP), jnp.float32),
            pltpu.VMEM((P, CP), jnp.float32),
            pltpu.VMEM((P, CP), jnp.float32),
            pltpu.SemaphoreType.DMA,
        ],
    )(xt, gx, gy)


def kernel(x, y):
    gx = y[..., 0].reshape(B, QB)
    gy = y[..., 1].reshape(B, QB)
    xts = [_transpose_batch(x[b]) for b in range(B)]
    outs = [_sc_gather(xts[b], gx[b], gy[b]) for b in range(B)]
    out = jnp.stack(outs)
    return out.reshape(B, N, S, CP)[..., :C]


# speedup vs baseline: 1.1364x; 1.1364x over previous
import functools

import jax
import jax.numpy as jnp
from jax import lax
from jax.experimental import pallas as pl
from jax.experimental.pallas import tpu as pltpu
from jax.experimental.pallas import tpu_sc as plsc

B, C, H, W = 4, 96, 384, 384
CP = 128
N, S = 4096, 16
QB = N * S
NC, NS, L = 2, 16, 16
NW = NC * NS
QPW = QB // NW
P = 128
NCHUNK = QPW // P
CL = C // L

HG = 8
NH = H // HG


def _tr_body(x_ref, o_ref):
    row = lax.broadcasted_iota(jnp.int32, (C, CP), 0)
    col = lax.broadcasted_iota(jnp.int32, (C, CP), 1)
    eye = (row == col).astype(jnp.float32)
    for j in range(HG):
        a = x_ref[0, :, j, :]
        t = lax.dot_general(a, eye, (((0,), (0,)), ((), ())),
                            preferred_element_type=jnp.float32,
                            precision=lax.Precision.HIGHEST)
        o_ref[pl.ds(j * W, W), :] = t


def _transpose_batch(x, b):
    return pl.pallas_call(
        _tr_body,
        out_shape=jax.ShapeDtypeStruct((H * W, CP), jnp.float32),
        grid=(NH,),
        in_specs=[pl.BlockSpec((1, C, HG, W), lambda i, b=b: (b, 0, i, 0))],
        out_specs=pl.BlockSpec((HG * W, CP), lambda i: (i, 0)),
    )(x)


def _sc_body(xt, gx, gy, out,
             gx_v, gy_v,
             w00_v, w01_v, w10_v, w11_v,
             i00_v, i01_v, i10_v, i11_v,
             r00, r01, r10, r11, out_v, sem):
    cid = lax.axis_index("c")
    sid = lax.axis_index("s")
    wid = sid * NC + cid
    qbase = wid * QPW

    pltpu.sync_copy(gx.at[pl.ds(qbase, QPW)], gx_v)
    pltpu.sync_copy(gy.at[pl.ds(qbase, QPW)], gy_v)

    def chunk(k, _):
        off = k * P

        def grp(j, _):
            sl = pl.ds(off + j * L, L)
            dst = pl.ds(j * L, L)
            px = (gx_v[sl] + 1.0) * (0.5 * (W - 1))
            py = (gy_v[sl] + 1.0) * (0.5 * (H - 1))
            px = jnp.minimum(jnp.maximum(px, 0.0), float(W - 1))
            py = jnp.minimum(jnp.maximum(py, 0.0), float(H - 1))
            x0 = px.astype(jnp.int32)
            y0 = py.astype(jnp.int32)
            fx = px - x0.astype(jnp.float32)
            fy = py - y0.astype(jnp.float32)
            dx = jnp.minimum(x0 + 1, W - 1) - x0
            dy = jnp.minimum(y0 + 1, H - 1) - y0
            i00 = y0 * W + x0
            i10 = i00 + dy * W
            gx1 = 1.0 - fx
            gy1 = 1.0 - fy
            w00_v[dst] = gx1 * gy1
            w01_v[dst] = fx * gy1
            w10_v[dst] = gx1 * fy
            w11_v[dst] = fx * fy
            i00_v[dst] = i00
            i01_v[dst] = i00 + dx
            i10_v[dst] = i10
            i11_v[dst] = i10 + dx
            return 0

        lax.fori_loop(0, P // L, grp, 0)

        cp0 = pltpu.async_copy(xt.at[i00_v], r00, sem)
        cp1 = pltpu.async_copy(xt.at[i01_v], r01, sem)
        cp2 = pltpu.async_copy(xt.at[i10_v], r10, sem)
        cp3 = pltpu.async_copy(xt.at[i11_v], r11, sem)
        cp0.wait()
        cp1.wait()
        cp2.wait()
        cp3.wait()

        def blend(g, _):
            gsl = pl.ds(g * L, L)
            w00 = w00_v[gsl]
            w01 = w01_v[gsl]
            w10 = w10_v[gsl]
            w11 = w11_v[gsl]
            for ii in range(L):
                i = g * L + ii
                a = w00[ii]
                b = w01[ii]
                c = w10[ii]
                d = w11[ii]
                for cc in range(CL):
                    sl = pl.ds(cc * L, L)
                    out_v[i, sl] = (a * r00[i, sl] + b * r01[i, sl]
                                    + c * r10[i, sl] + d * r11[i, sl])
            return 0

        lax.fori_loop(0, P // L, blend, 0)

        pltpu.sync_copy(out_v, out.at[pl.ds(qbase + off, P)])
        return 0

    lax.fori_loop(0, NCHUNK, chunk, 0)


def _sc_gather(xt, gx, gy):
    mesh = plsc.VectorSubcoreMesh(core_axis_name="c", subcore_axis_name="s")
    return pl.kernel(
        _sc_body,
        out_type=jax.ShapeDtypeStruct((QB, CP), jnp.float32),
        mesh=mesh,
        compiler_params=pltpu.CompilerParams(use_tc_tiling_on_sc=True),
        scratch_types=[
            pltpu.VMEM((QPW,), jnp.float32),
            pltpu.VMEM((QPW,), jnp.float32),
            pltpu.VMEM((P,), jnp.float32),
            pltpu.VMEM((P,), jnp.float32),
            pltpu.VMEM((P,), jnp.float32),
            pltpu.VMEM((P,), jnp.float32),
            pltpu.VMEM((P,), jnp.int32),
            pltpu.VMEM((P,), jnp.int32),
            pltpu.VMEM((P,), jnp.int32),
            pltpu.VMEM((P,), jnp.int32),
            pltpu.VMEM((P, CP), jnp.float32),
            pltpu.VMEM((P, CP), jnp.float32),
            pltpu.VMEM((P, CP), jnp.float32),
            pltpu.VMEM((P, CP), jnp.float32),
            pltpu.VMEM((P, CP), jnp.float32),
            pltpu.SemaphoreType.DMA,
        ],
    )(xt, gx, gy)


def kernel(x, y):
    gx = y[..., 0].reshape(B, QB)
    gy = y[..., 1].reshape(B, QB)
    xts = [_transpose_batch(x, b) for b in range(B)]
    outs = [_sc_gather(xts[b], gx[b], gy[b]) for b in range(B)]
    out = jnp.stack(outs)
    return out.reshape(B, N, S, CP)[..., :C]


# speedup vs baseline: 1.3936x; 1.2263x over previous
import functools

import jax
import jax.numpy as jnp
from jax import lax
from jax.experimental import pallas as pl
from jax.experimental.pallas import tpu as pltpu
from jax.experimental.pallas import tpu_sc as plsc

B, C, H, W = 4, 96, 384, 384
CP = 128
N, S = 4096, 16
QB = N * S
NC, NS, L = 2, 16, 16
NW = NC * NS
QPW = QB // NW
P = 64
NCHUNK = QPW // P
NPAIR = NCHUNK // 2
CL = C // L

HG = 8
NH = H // HG


def _tr_body(x_ref, o_ref):
    row = lax.broadcasted_iota(jnp.int32, (C, CP), 0)
    col = lax.broadcasted_iota(jnp.int32, (C, CP), 1)
    eye = (row == col).astype(jnp.float32)
    for j in range(HG):
        a = x_ref[0, :, j, :]
        t = lax.dot_general(a, eye, (((0,), (0,)), ((), ())),
                            preferred_element_type=jnp.float32,
                            precision=lax.Precision.HIGHEST)
        o_ref[pl.ds(j * W, W), :] = t


def _transpose_batch(x, b):
    return pl.pallas_call(
        _tr_body,
        out_shape=jax.ShapeDtypeStruct((H * W, CP), jnp.float32),
        grid=(NH,),
        in_specs=[pl.BlockSpec((1, C, HG, W), lambda i, b=b: (b, 0, i, 0))],
        out_specs=pl.BlockSpec((HG * W, CP), lambda i: (i, 0)),
    )(x)


def _sc_body(xt, gx, gy, out,
             gx_v, gy_v,
             w00_v, w01_v, w10_v, w11_v,
             i00_v, i01_v, i10_v, i11_v,
             r00, r01, r10, r11, out_v,
             sem0, sem1):
    cid = lax.axis_index("c")
    sid = lax.axis_index("s")
    wid = sid * NC + cid
    qbase = wid * QPW
    sems = (sem0, sem1)

    pltpu.sync_copy(gx.at[pl.ds(qbase, QPW)], gx_v)
    pltpu.sync_copy(gy.at[pl.ds(qbase, QPW)], gy_v)

    def compute_idx(off, ph):
        def grp(j, _):
            sl = pl.ds(off + j * L, L)
            dst = pl.ds(j * L, L)
            px = (gx_v[sl] + 1.0) * (0.5 * (W - 1))
            py = (gy_v[sl] + 1.0) * (0.5 * (H - 1))
            px = jnp.minimum(jnp.maximum(px, 0.0), float(W - 1))
            py = jnp.minimum(jnp.maximum(py, 0.0), float(H - 1))
            x0 = px.astype(jnp.int32)
            y0 = py.astype(jnp.int32)
            fx = px - x0.astype(jnp.float32)
            fy = py - y0.astype(jnp.float32)
            dx = jnp.minimum(x0 + 1, W - 1) - x0
            dy = jnp.minimum(y0 + 1, H - 1) - y0
            i00 = y0 * W + x0
            i10 = i00 + dy * W
            gx1 = 1.0 - fx
            gy1 = 1.0 - fy
            w00_v[ph, dst] = gx1 * gy1
            w01_v[ph, dst] = fx * gy1
            w10_v[ph, dst] = gx1 * fy
            w11_v[ph, dst] = fx * fy
            i00_v[ph, dst] = i00
            i01_v[ph, dst] = i00 + dx
            i10_v[ph, dst] = i10
            i11_v[ph, dst] = i10 + dx
            return 0

        lax.fori_loop(0, P // L, grp, 0)

    def issue(ph):
        pltpu.async_copy(xt.at[i00_v.at[ph]], r00.at[ph], sems[ph])
        pltpu.async_copy(xt.at[i01_v.at[ph]], r01.at[ph], sems[ph])
        pltpu.async_copy(xt.at[i10_v.at[ph]], r10.at[ph], sems[ph])
        pltpu.async_copy(xt.at[i11_v.at[ph]], r11.at[ph], sems[ph])

    def drain(ph):
        for rr in (r00, r01, r10, r11):
            pltpu.make_async_copy(xt.at[i00_v.at[ph]], rr.at[ph],
                                  sems[ph]).wait()

    def blend(ph):
        def bgrp(g, _):
            gsl = pl.ds(g * L, L)
            w00 = w00_v[ph, gsl]
            w01 = w01_v[ph, gsl]
            w10 = w10_v[ph, gsl]
            w11 = w11_v[ph, gsl]
            for ii in range(L):
                i = g * L + ii
                a = w00[ii]
                b = w01[ii]
                c = w10[ii]
                d = w11[ii]
                for cc in range(CL):
                    sl = pl.ds(cc * L, L)
                    out_v[ph, i, sl] = (a * r00[ph, i, sl]
                                        + b * r01[ph, i, sl]
                                        + c * r10[ph, i, sl]
                                        + d * r11[ph, i, sl])
            return 0

        lax.fori_loop(0, P // L, bgrp, 0)

    def flush(off, ph):
        pltpu.sync_copy(out_v.at[ph], out.at[pl.ds(qbase + off, P)])

    compute_idx(0, 0)
    issue(0)

    def pair(m, _):
        k0 = m * 2
        k1 = k0 + 1
        compute_idx(k1 * P, 1)
        issue(1)
        drain(0)
        blend(0)
        flush(k0 * P, 0)

        @pl.when(m < NPAIR - 1)
        def _():
            compute_idx((k1 + 1) * P, 0)
            issue(0)

        drain(1)
        blend(1)
        flush(k1 * P, 1)
        return 0

    lax.fori_loop(0, NPAIR, pair, 0)


def _sc_gather(xt, gx, gy):
    mesh = plsc.VectorSubcoreMesh(core_axis_name="c", subcore_axis_name="s")
    return pl.kernel(
        _sc_body,
        out_type=jax.ShapeDtypeStruct((QB, CP), jnp.float32),
        mesh=mesh,
        compiler_params=pltpu.CompilerParams(use_tc_tiling_on_sc=True),
        scratch_types=[
            pltpu.VMEM((QPW,), jnp.float32),
            pltpu.VMEM((QPW,), jnp.float32),
            pltpu.VMEM((2, P), jnp.float32),
            pltpu.VMEM((2, P), jnp.float32),
            pltpu.VMEM((2, P), jnp.float32),
            pltpu.VMEM((2, P), jnp.float32),
            pltpu.VMEM((2, P), jnp.int32),
            pltpu.VMEM((2, P), jnp.int32),
            pltpu.VMEM((2, P), jnp.int32),
            pltpu.VMEM((2, P), jnp.int32),
            pltpu.VMEM((2, P, CP), jnp.float32),
            pltpu.VMEM((2, P, CP), jnp.float32),
            pltpu.VMEM((2, P, CP), jnp.float32),
            pltpu.VMEM((2, P, CP), jnp.float32),
            pltpu.VMEM((2, P, CP), jnp.float32),
            pltpu.SemaphoreType.DMA,
            pltpu.SemaphoreType.DMA,
        ],
    )(xt, gx, gy)


def kernel(x, y):
    gx = y[..., 0].reshape(B, QB)
    gy = y[..., 1].reshape(B, QB)
    xts = [_transpose_batch(x, b) for b in range(B)]
    outs = [_sc_gather(xts[b], gx[b], gy[b]) for b in range(B)]
    out = jnp.stack(outs)
    return out.reshape(B, N, S, CP)[..., :C]
